# Initial kernel scaffold; baseline (speedup 1.0000x reference)
import jax
import jax.numpy as jnp
from jax.experimental import pallas as pl


def kernel(x, edge_index, batch, W_l, b_l, W_r, W_lin, b_lin):
    raise NotImplementedError("write your pallas kernel here")


# speedup vs baseline: 160.0849x; 160.0849x over previous
import functools

import jax
import jax.numpy as jnp
from jax import lax
from jax.experimental import pallas as pl
from jax.experimental.pallas import tpu as pltpu
from jax.experimental.pallas import tpu_sc as plsc

NUM_GRAPHS = 256

_info = plsc.get_sparse_core_info()
_NC, _NS, _L = _info.num_cores, _info.num_subcores, _info.num_lanes
_NW = _NC * _NS


def _worker_id():
    return lax.axis_index("s") * _NC + lax.axis_index("c")


def _make_gather(N, E, CH):
    EPW = E // _NW
    mesh = plsc.VectorSubcoreMesh(core_axis_name="c", subcore_axis_name="s")

    @functools.partial(
        pl.kernel, mesh=mesh,
        out_type=jax.ShapeDtypeStruct((E,), jnp.float32),
        scratch_types=[
            pltpu.VMEM((N,), jnp.float32),
            pltpu.VMEM((CH,), jnp.int32),
            pltpu.VMEM((CH,), jnp.float32),
        ],
        compiler_params=pltpu.CompilerParams(needs_layout_passes=False),
    )
    def gather_k(x_hbm, edge_hbm, vals_hbm, x_v, src_v, vals_v):
        base = _worker_id() * EPW
        pltpu.sync_copy(x_hbm, x_v)

        def chunk(ci, carry):
            off = base + ci * CH
            pltpu.sync_copy(edge_hbm.at[pl.ds(off, CH)], src_v)

            def vec(i, c):
                idx = src_v[pl.ds(i * _L, _L)]
                vals_v[pl.ds(i * _L, _L)] = plsc.load_gather(x_v, [idx])
                return c

            lax.fori_loop(0, CH // _L, vec, 0)
            pltpu.sync_copy(vals_v, vals_hbm.at[pl.ds(off, CH)])
            return carry

        lax.fori_loop(0, EPW // CH, chunk, 0)

    return gather_k


def _make_scatter(N_pad, E, CH):
    EPW = E // _NW
    mesh = plsc.VectorSubcoreMesh(core_axis_name="c", subcore_axis_name="s")

    @functools.partial(
        pl.kernel, mesh=mesh,
        out_type=jax.ShapeDtypeStruct((_NW, N_pad), jnp.float32),
        scratch_types=[
            pltpu.VMEM((N_pad,), jnp.float32),
            pltpu.VMEM((CH,), jnp.int32),
            pltpu.VMEM((CH,), jnp.float32),
        ],
        compiler_params=pltpu.CompilerParams(needs_layout_passes=False),
    )
    def scatter_k(edge_hbm, vals_hbm, part_hbm, acc_v, dst_v, vals_v):
        wid = _worker_id()
        base = wid * EPW

        def zero(i, c):
            acc_v[pl.ds(i * _L, _L)] = jnp.zeros((_L,), jnp.float32)
            return c

        lax.fori_loop(0, N_pad // _L, zero, 0)

        def chunk(ci, carry):
            off = base + ci * CH
            pltpu.sync_copy(edge_hbm.at[pl.ds(E + off, CH)], dst_v)
            pltpu.sync_copy(vals_hbm.at[pl.ds(off, CH)], vals_v)

            def vec(i, c):
                idx = dst_v[pl.ds(i * _L, _L)]
                v = vals_v[pl.ds(i * _L, _L)]
                plsc.addupdate_scatter(acc_v, [idx], v)
                return c

            lax.fori_loop(0, CH // _L, vec, 0)
            return carry

        lax.fori_loop(0, EPW // CH, chunk, 0)
        pltpu.sync_copy(acc_v, part_hbm.at[wid])

    return scatter_k


def _tc_body(nblk, part_ref, x_ref, batch_ref, wl_ref, bl_ref, wr_ref,
             wlin_ref, blin_ref, out_ref, sums, counts):
    i = pl.program_id(0)

    @pl.when(i == 0)
    def _init():
        sums[...] = jnp.zeros_like(sums)
        counts[...] = jnp.zeros_like(counts)

    agg = jnp.sum(part_ref[...], axis=0)
    xb = x_ref[0, :]
    h = jnp.maximum(
        agg[:, None] * wl_ref[0, :][None, :]
        + xb[:, None] * wr_ref[0, :][None, :]
        + bl_ref[0, :][None, :], 0.0)
    bb = batch_ref[0, :]
    gid = lax.broadcasted_iota(jnp.int32, (NUM_GRAPHS, 1), 0)
    onehot = (bb[None, :] == gid).astype(jnp.bfloat16)
    sums[...] += jnp.dot(onehot, h.astype(jnp.bfloat16),
                         preferred_element_type=jnp.float32)
    cnt = jnp.sum(onehot.astype(jnp.float32), axis=1, keepdims=True)
    counts[...] += jnp.broadcast_to(cnt, counts.shape)

    @pl.when(i == nblk - 1)
    def _fin():
        pooled = sums[...] / jnp.maximum(counts[...], 1.0)
        out_ref[...] = (jnp.dot(pooled, wlin_ref[...],
                                preferred_element_type=jnp.float32)
                        + blin_ref[0, :][None, :])


def _make_tc(N, H, OUT, B):
    nblk = N // B
    return pl.pallas_call(
        functools.partial(_tc_body, nblk),
        grid=(nblk,),
        in_specs=[
            pl.BlockSpec((_NW, B), lambda i: (0, i)),
            pl.BlockSpec((1, B), lambda i: (0, i)),
            pl.BlockSpec((1, B), lambda i: (0, i)),
            pl.BlockSpec((1, H), lambda i: (0, 0)),
            pl.BlockSpec((1, H), lambda i: (0, 0)),
            pl.BlockSpec((1, H), lambda i: (0, 0)),
            pl.BlockSpec((H, OUT), lambda i: (0, 0)),
            pl.BlockSpec((1, OUT), lambda i: (0, 0)),
        ],
        out_specs=pl.BlockSpec((NUM_GRAPHS, OUT), lambda i: (0, 0)),
        out_shape=jax.ShapeDtypeStruct((NUM_GRAPHS, OUT), jnp.float32),
        scratch_shapes=[
            pltpu.VMEM((NUM_GRAPHS, H), jnp.float32),
            pltpu.VMEM((NUM_GRAPHS, H), jnp.float32),
        ],
    )


def kernel(x, edge_index, batch, W_l, b_l, W_r, W_lin, b_lin):
    N = x.shape[0]
    E = edge_index.shape[1]
    H = W_l.shape[0]
    OUT = W_lin.shape[0]
    CH = 4000
    B = 4096
    N_pad = -(-N // B) * B

    xf = x.reshape(N)
    edge_flat = edge_index.reshape(2 * E)
    vals = _make_gather(N, E, CH)(xf, edge_flat)
    partials = _make_scatter(N_pad, E, CH)(edge_flat, vals)
    x_pad = jnp.pad(x.reshape(1, N), ((0, 0), (0, N_pad - N)))
    batch_pad = jnp.pad(batch.reshape(1, N), ((0, 0), (0, N_pad - N)),
                        constant_values=NUM_GRAPHS)
    out = _make_tc(N_pad, H, OUT, B)(
        partials,
        x_pad,
        batch_pad,
        W_l.reshape(1, H),
        b_l.reshape(1, H),
        W_r.reshape(1, H),
        W_lin.T,
        b_lin.reshape(1, OUT),
    )
    return out


# speedup vs baseline: 223.6893x; 1.3973x over previous
import functools

import jax
import jax.numpy as jnp
from jax import lax
from jax.experimental import pallas as pl
from jax.experimental.pallas import tpu as pltpu
from jax.experimental.pallas import tpu_sc as plsc

NUM_GRAPHS = 256

_info = plsc.get_sparse_core_info()
_NC, _NS, _L = _info.num_cores, _info.num_subcores, _info.num_lanes
_NW = _NC * _NS

_SC_PARAMS = pltpu.CompilerParams(needs_layout_passes=False)


def _worker_id():
    return lax.axis_index("s") * _NC + lax.axis_index("c")


def _make_gather(N, E, CH):
    EPW = E // _NW
    NCH = EPW // CH
    mesh = plsc.VectorSubcoreMesh(core_axis_name="c", subcore_axis_name="s")

    @functools.partial(
        pl.kernel, mesh=mesh,
        out_type=jax.ShapeDtypeStruct((E,), jnp.float32),
        scratch_types=[
            pltpu.VMEM((N,), jnp.float32),
            pltpu.VMEM((CH,), jnp.int32),
            pltpu.VMEM((CH,), jnp.int32),
            pltpu.VMEM((CH,), jnp.float32),
            pltpu.VMEM((CH,), jnp.float32),
            pltpu.SemaphoreType.DMA,
            pltpu.SemaphoreType.DMA,
            pltpu.SemaphoreType.DMA,
            pltpu.SemaphoreType.DMA,
            pltpu.SemaphoreType.DMA,
        ],
        compiler_params=_SC_PARAMS,
    )
    def gather_k(x_hbm, edge_hbm, vals_hbm, x_v, s0, s1, v0, v1,
                 sem_x, sem_s0, sem_s1, sem_v0, sem_v1):
        base = _worker_id() * EPW
        pltpu.async_copy(x_hbm, x_v, sem_x)
        pltpu.async_copy(edge_hbm.at[pl.ds(base, CH)], s0, sem_s0)
        pltpu.make_async_copy(x_hbm, x_v, sem_x).wait()

        def do_chunk(off, sbuf, vbuf):
            def vec(i, c):
                idx = sbuf[pl.ds(i * _L, _L)]
                vbuf[pl.ds(i * _L, _L)] = plsc.load_gather(x_v, [idx])
                return c
            lax.fori_loop(0, CH // _L, vec, 0, unroll=10)

        def pair(p, carry):
            c0 = 2 * p
            off0 = base + c0 * CH
            off1 = off0 + CH

            @pl.when(c0 + 1 < NCH)
            def _():
                pltpu.async_copy(edge_hbm.at[pl.ds(off1, CH)], s1, sem_s1)
            pltpu.make_async_copy(edge_hbm.at[pl.ds(off0, CH)], s0,
                                  sem_s0).wait()
            @pl.when(c0 >= 2)
            def _():
                pltpu.make_async_copy(v0, vals_hbm.at[pl.ds(off0, CH)],
                                      sem_v0).wait()
            do_chunk(off0, s0, v0)
            pltpu.async_copy(v0, vals_hbm.at[pl.ds(off0, CH)], sem_v0)

            @pl.when(c0 + 2 < NCH)
            def _():
                pltpu.async_copy(edge_hbm.at[pl.ds(off0 + 2 * CH, CH)], s0,
                                 sem_s0)
            pltpu.make_async_copy(edge_hbm.at[pl.ds(off1, CH)], s1,
                                  sem_s1).wait()
            @pl.when(c0 >= 2)
            def _():
                pltpu.make_async_copy(v1, vals_hbm.at[pl.ds(off1, CH)],
                                      sem_v1).wait()
            do_chunk(off1, s1, v1)
            pltpu.async_copy(v1, vals_hbm.at[pl.ds(off1, CH)], sem_v1)
            return carry

        lax.fori_loop(0, NCH // 2, pair, 0)
        pltpu.make_async_copy(v0, vals_hbm.at[pl.ds(base, CH)], sem_v0).wait()
        pltpu.make_async_copy(v1, vals_hbm.at[pl.ds(base, CH)], sem_v1).wait()

    return gather_k


def _make_scatter(N_pad, E, CH):
    EPW = E // _NW
    NCH = EPW // CH
    mesh = plsc.VectorSubcoreMesh(core_axis_name="c", subcore_axis_name="s")

    @functools.partial(
        pl.kernel, mesh=mesh,
        out_type=jax.ShapeDtypeStruct((_NW, N_pad), jnp.float32),
        scratch_types=[
            pltpu.VMEM((N_pad,), jnp.float32),
            pltpu.VMEM((CH,), jnp.int32),
            pltpu.VMEM((CH,), jnp.int32),
            pltpu.VMEM((CH,), jnp.float32),
            pltpu.VMEM((CH,), jnp.float32),
            pltpu.SemaphoreType.DMA,
            pltpu.SemaphoreType.DMA,
            pltpu.SemaphoreType.DMA,
            pltpu.SemaphoreType.DMA,
        ],
        compiler_params=_SC_PARAMS,
    )
    def scatter_k(edge_hbm, vals_hbm, part_hbm, acc_v, d0, d1, v0, v1,
                  sem_d0, sem_d1, sem_v0, sem_v1):
        wid = _worker_id()
        base = wid * EPW

        pltpu.async_copy(edge_hbm.at[pl.ds(E + base, CH)], d0, sem_d0)
        pltpu.async_copy(vals_hbm.at[pl.ds(base, CH)], v0, sem_v0)

        def zero(i, c):
            for k in range(8):
                acc_v[pl.ds(i * 8 * _L + k * _L, _L)] = jnp.zeros(
                    (_L,), jnp.float32)
            return c

        lax.fori_loop(0, N_pad // (8 * _L), zero, 0)

        def do_chunk(dbuf, vbuf):
            def vec(i, c):
                idx = dbuf[pl.ds(i * _L, _L)]
                v = vbuf[pl.ds(i * _L, _L)]
                plsc.addupdate_scatter(acc_v, [idx], v)
                return c
            lax.fori_loop(0, CH // _L, vec, 0, unroll=10)

        def pair(p, carry):
            c0 = 2 * p
            off0 = base + c0 * CH
            off1 = off0 + CH

            @pl.when(c0 + 1 < NCH)
            def _():
                pltpu.async_copy(edge_hbm.at[pl.ds(E + off1, CH)], d1, sem_d1)
                pltpu.async_copy(vals_hbm.at[pl.ds(off1, CH)], v1, sem_v1)
            pltpu.make_async_copy(edge_hbm.at[pl.ds(E + off0, CH)], d0,
                                  sem_d0).wait()
            pltpu.make_async_copy(vals_hbm.at[pl.ds(off0, CH)], v0,
                                  sem_v0).wait()
            do_chunk(d0, v0)

            @pl.when(c0 + 2 < NCH)
            def _():
                pltpu.async_copy(edge_hbm.at[pl.ds(E + off0 + 2 * CH, CH)],
                                 d0, sem_d0)
                pltpu.async_copy(vals_hbm.at[pl.ds(off0 + 2 * CH, CH)], v0,
                                 sem_v0)
            pltpu.make_async_copy(edge_hbm.at[pl.ds(E + off1, CH)], d1,
                                  sem_d1).wait()
            pltpu.make_async_copy(vals_hbm.at[pl.ds(off1, CH)], v1,
                                  sem_v1).wait()
            do_chunk(d1, v1)
            return carry

        lax.fori_loop(0, NCH // 2, pair, 0)
        pltpu.sync_copy(acc_v, part_hbm.at[wid])

    return scatter_k


def _tc_body(nblk, part_ref, x_ref, batch_ref, wl_ref, bl_ref, wr_ref,
             wlin_ref, blin_ref, out_ref, sums, counts):
    i = pl.program_id(0)

    @pl.when(i == 0)
    def _init():
        sums[...] = jnp.zeros_like(sums)
        counts[...] = jnp.zeros_like(counts)

    agg = jnp.sum(part_ref[...], axis=0)
    xb = x_ref[0, :]
    h = jnp.maximum(
        agg[:, None] * wl_ref[0, :][None, :]
        + xb[:, None] * wr_ref[0, :][None, :]
        + bl_ref[0, :][None, :], 0.0)
    bb = batch_ref[0, :]
    gid = lax.broadcasted_iota(jnp.int32, (NUM_GRAPHS, 1), 0)
    onehot = (bb[None, :] == gid).astype(jnp.bfloat16)
    sums[...] += jnp.dot(onehot, h.astype(jnp.bfloat16),
                         preferred_element_type=jnp.float32)
    cnt = jnp.sum(onehot.astype(jnp.float32), axis=1, keepdims=True)
    counts[...] += jnp.broadcast_to(cnt, counts.shape)

    @pl.when(i == nblk - 1)
    def _fin():
        pooled = sums[...] / jnp.maximum(counts[...], 1.0)
        out_ref[...] = (jnp.dot(pooled, wlin_ref[...],
                                preferred_element_type=jnp.float32)
                        + blin_ref[0, :][None, :])


def _make_tc(N, H, OUT, B):
    nblk = N // B
    return pl.pallas_call(
        functools.partial(_tc_body, nblk),
        grid=(nblk,),
        in_specs=[
            pl.BlockSpec((_NW, B), lambda i: (0, i)),
            pl.BlockSpec((1, B), lambda i: (0, i)),
            pl.BlockSpec((1, B), lambda i: (0, i)),
            pl.BlockSpec((1, H), lambda i: (0, 0)),
            pl.BlockSpec((1, H), lambda i: (0, 0)),
            pl.BlockSpec((1, H), lambda i: (0, 0)),
            pl.BlockSpec((H, OUT), lambda i: (0, 0)),
            pl.BlockSpec((1, OUT), lambda i: (0, 0)),
        ],
        out_specs=pl.BlockSpec((NUM_GRAPHS, OUT), lambda i: (0, 0)),
        out_shape=jax.ShapeDtypeStruct((NUM_GRAPHS, OUT), jnp.float32),
        scratch_shapes=[
            pltpu.VMEM((NUM_GRAPHS, H), jnp.float32),
            pltpu.VMEM((NUM_GRAPHS, H), jnp.float32),
        ],
    )


def kernel(x, edge_index, batch, W_l, b_l, W_r, W_lin, b_lin):
    N = x.shape[0]
    E = edge_index.shape[1]
    H = W_l.shape[0]
    OUT = W_lin.shape[0]
    CH = 4000
    B = 4096
    N_pad = -(-N // B) * B

    xf = x.reshape(N)
    edge_flat = edge_index.reshape(2 * E)
    vals = _make_gather(N, E, CH)(xf, edge_flat)
    partials = _make_scatter(N_pad, E, CH)(edge_flat, vals)
    x_pad = jnp.pad(x.reshape(1, N), ((0, 0), (0, N_pad - N)))
    batch_pad = jnp.pad(batch.reshape(1, N), ((0, 0), (0, N_pad - N)),
                        constant_values=NUM_GRAPHS)
    out = _make_tc(N_pad, H, OUT, B)(
        partials,
        x_pad,
        batch_pad,
        W_l.reshape(1, H),
        b_l.reshape(1, H),
        W_r.reshape(1, H),
        W_lin.T,
        b_lin.reshape(1, OUT),
    )
    return out


# speedup vs baseline: 340.7180x; 1.5232x over previous
import functools

import jax
import jax.numpy as jnp
from jax import lax
from jax.experimental import pallas as pl
from jax.experimental.pallas import tpu as pltpu
from jax.experimental.pallas import tpu_sc as plsc

NUM_GRAPHS = 256

_info = plsc.get_sparse_core_info()
_NC, _NS, _L = _info.num_cores, _info.num_subcores, _info.num_lanes
_NW = _NC * _NS

_SC_PARAMS = pltpu.CompilerParams(needs_layout_passes=False)


def _worker_id():
    return lax.axis_index("s") * _NC + lax.axis_index("c")


def _make_edge_kernel(N, N_pad, E, CH):
    NCHT = E // CH
    NV = CH // _L
    mesh = plsc.VectorSubcoreMesh(core_axis_name="c", subcore_axis_name="s")

    @functools.partial(
        pl.kernel, mesh=mesh,
        out_type=(
            jax.ShapeDtypeStruct((_NW, N_pad), jnp.float32),
            jax.ShapeDtypeStruct((E,), jnp.float32),
        ),
        scratch_types=[
            pltpu.VMEM((N_pad,), jnp.float32),
            pltpu.VMEM((2, CH), jnp.int32),
            pltpu.VMEM((2, CH), jnp.int32),
            pltpu.VMEM((CH,), jnp.float32),
            pltpu.VMEM((CH,), jnp.float32),
            pltpu.SemaphoreType.DMA,
            pltpu.SemaphoreType.DMA,
            pltpu.SemaphoreType.DMA,
            pltpu.SemaphoreType.DMA,
            pltpu.SemaphoreType.DMA,
        ],
        compiler_params=_SC_PARAMS,
    )
    def edge_k(x_hbm, edge_hbm, part_hbm, vals_hbm, big, e0, e1, v0, v1,
               sem_x, sem_e0, sem_e1, sem_v0, sem_v1):
        wid = _worker_id()
        cnt = (NCHT - wid + _NW - 1) // _NW

        def col(j):
            return (wid + j * _NW) * CH

        pltpu.async_copy(x_hbm, big.at[pl.ds(0, N)], sem_x)
        pltpu.async_copy(edge_hbm.at[:, pl.ds(col(0), CH)], e0, sem_e0)
        pltpu.make_async_copy(x_hbm, big.at[pl.ds(0, N)], sem_x).wait()

        def gather_chunk(ebuf, vbuf):
            @plsc.parallel_loop(0, NV, unroll=8)
            def _(i):
                idx = ebuf[0, pl.ds(i * _L, _L)]
                vbuf[pl.ds(i * _L, _L)] = plsc.load_gather(big, [idx])

        def g_pair(p, carry):
            j0 = 2 * p
            j1 = j0 + 1

            @pl.when(j1 < cnt)
            def _():
                pltpu.async_copy(edge_hbm.at[:, pl.ds(col(j1), CH)], e1,
                                 sem_e1)
            pltpu.make_async_copy(edge_hbm.at[:, pl.ds(col(j0), CH)], e0,
                                  sem_e0).wait()
            @pl.when(j0 >= 2)
            def _():
                pltpu.make_async_copy(v0, vals_hbm.at[pl.ds(col(j0), CH)],
                                      sem_v0).wait()
            gather_chunk(e0, v0)
            pltpu.async_copy(v0, vals_hbm.at[pl.ds(col(j0), CH)], sem_v0)

            @pl.when(j0 + 2 < cnt)
            def _():
                pltpu.async_copy(edge_hbm.at[:, pl.ds(col(j0 + 2), CH)], e0,
                                 sem_e0)

            @pl.when(j1 < cnt)
            def _():
                pltpu.make_async_copy(edge_hbm.at[:, pl.ds(col(j1), CH)], e1,
                                      sem_e1).wait()
                @pl.when(j1 >= 2)
                def _():
                    pltpu.make_async_copy(
                        v1, vals_hbm.at[pl.ds(col(j1), CH)], sem_v1).wait()
                gather_chunk(e1, v1)
                pltpu.async_copy(v1, vals_hbm.at[pl.ds(col(j1), CH)], sem_v1)
            return carry

        lax.fori_loop(0, (cnt + 1) // 2, g_pair, 0)
        pltpu.make_async_copy(v0, vals_hbm.at[pl.ds(0, CH)], sem_v0).wait()

        @pl.when(cnt >= 2)
        def _():
            pltpu.make_async_copy(v1, vals_hbm.at[pl.ds(0, CH)], sem_v1).wait()

        pltpu.async_copy(edge_hbm.at[:, pl.ds(col(0), CH)], e0, sem_e0)
        pltpu.async_copy(vals_hbm.at[pl.ds(col(0), CH)], v0, sem_v0)

        def zero(i, c):
            for k in range(8):
                big[pl.ds(i * 8 * _L + k * _L, _L)] = jnp.zeros(
                    (_L,), jnp.float32)
            return c

        lax.fori_loop(0, N_pad // (8 * _L), zero, 0)

        def scatter_chunk(ebuf, vbuf):
            def vec(i, c):
                idx = ebuf[1, pl.ds(i * _L, _L)]
                v = vbuf[pl.ds(i * _L, _L)]
                plsc.addupdate_scatter(big, [idx], v)
                return c
            lax.fori_loop(0, NV, vec, 0, unroll=8)

        def s_pair(p, carry):
            j0 = 2 * p
            j1 = j0 + 1

            @pl.when(j1 < cnt)
            def _():
                pltpu.async_copy(edge_hbm.at[:, pl.ds(col(j1), CH)], e1,
                                 sem_e1)
                pltpu.async_copy(vals_hbm.at[pl.ds(col(j1), CH)], v1, sem_v1)
            pltpu.make_async_copy(edge_hbm.at[:, pl.ds(col(j0), CH)], e0,
                                  sem_e0).wait()
            pltpu.make_async_copy(vals_hbm.at[pl.ds(col(j0), CH)], v0,
                                  sem_v0).wait()
            scatter_chunk(e0, v0)

            @pl.when(j0 + 2 < cnt)
            def _():
                pltpu.async_copy(edge_hbm.at[:, pl.ds(col(j0 + 2), CH)], e0,
                                 sem_e0)
                pltpu.async_copy(vals_hbm.at[pl.ds(col(j0 + 2), CH)], v0,
                                 sem_v0)

            @pl.when(j1 < cnt)
            def _():
                pltpu.make_async_copy(edge_hbm.at[:, pl.ds(col(j1), CH)], e1,
                                      sem_e1).wait()
                pltpu.make_async_copy(vals_hbm.at[pl.ds(col(j1), CH)], v1,
                                      sem_v1).wait()
                scatter_chunk(e1, v1)
            return carry

        lax.fori_loop(0, (cnt + 1) // 2, s_pair, 0)
        pltpu.sync_copy(big, part_hbm.at[wid])

    return edge_k


def _tc_body(nblk, part_ref, x_ref, batch_ref, wl_ref, bl_ref, wr_ref,
             wlin_ref, blin_ref, out_ref, sums, counts):
    i = pl.program_id(0)

    @pl.when(i == 0)
    def _init():
        sums[...] = jnp.zeros_like(sums)
        counts[...] = jnp.zeros_like(counts)

    agg = jnp.sum(part_ref[...], axis=0)
    xb = x_ref[0, :]
    h = jnp.maximum(
        agg[:, None] * wl_ref[0, :][None, :]
        + xb[:, None] * wr_ref[0, :][None, :]
        + bl_ref[0, :][None, :], 0.0)
    bb = batch_ref[0, :]
    gid = lax.broadcasted_iota(jnp.int32, (NUM_GRAPHS, 1), 0)
    onehot = (bb[None, :] == gid).astype(jnp.bfloat16)
    sums[...] += jnp.dot(onehot, h.astype(jnp.bfloat16),
                         preferred_element_type=jnp.float32)
    cnt = jnp.sum(onehot.astype(jnp.float32), axis=1, keepdims=True)
    counts[...] += jnp.broadcast_to(cnt, counts.shape)

    @pl.when(i == nblk - 1)
    def _fin():
        pooled = sums[...] / jnp.maximum(counts[...], 1.0)
        out_ref[...] = (jnp.dot(pooled, wlin_ref[...],
                                preferred_element_type=jnp.float32)
                        + blin_ref[0, :][None, :])


def _make_tc(N, H, OUT, B):
    nblk = N // B
    return pl.pallas_call(
        functools.partial(_tc_body, nblk),
        grid=(nblk,),
        in_specs=[
            pl.BlockSpec((_NW, B), lambda i: (0, i)),
            pl.BlockSpec((1, B), lambda i: (0, i)),
            pl.BlockSpec((1, B), lambda i: (0, i)),
            pl.BlockSpec((1, H), lambda i: (0, 0)),
            pl.BlockSpec((1, H), lambda i: (0, 0)),
            pl.BlockSpec((1, H), lambda i: (0, 0)),
            pl.BlockSpec((H, OUT), lambda i: (0, 0)),
            pl.BlockSpec((1, OUT), lambda i: (0, 0)),
        ],
        out_specs=pl.BlockSpec((NUM_GRAPHS, OUT), lambda i: (0, 0)),
        out_shape=jax.ShapeDtypeStruct((NUM_GRAPHS, OUT), jnp.float32),
        scratch_shapes=[
            pltpu.VMEM((NUM_GRAPHS, H), jnp.float32),
            pltpu.VMEM((NUM_GRAPHS, H), jnp.float32),
        ],
    )


def kernel(x, edge_index, batch, W_l, b_l, W_r, W_lin, b_lin):
    N = x.shape[0]
    E = edge_index.shape[1]
    H = W_l.shape[0]
    OUT = W_lin.shape[0]
    CH = 3200
    B = 4096
    N_pad = -(-N // B) * B

    xf = x.reshape(N)
    partials, _vals = _make_edge_kernel(N, N_pad, E, CH)(xf, edge_index)
    x_pad = jnp.pad(x.reshape(1, N), ((0, 0), (0, N_pad - N)))
    batch_pad = jnp.pad(batch.reshape(1, N), ((0, 0), (0, N_pad - N)),
                        constant_values=NUM_GRAPHS)
    out = _make_tc(N_pad, H, OUT, B)(
        partials,
        x_pad,
        batch_pad,
        W_l.reshape(1, H),
        b_l.reshape(1, H),
        W_r.reshape(1, H),
        W_lin.T,
        b_lin.reshape(1, OUT),
    )
    return out
